# Initial kernel scaffold; baseline (speedup 1.0000x reference)
import jax
import jax.numpy as jnp
from jax.experimental import pallas as pl


def kernel(x_0, x_1, edge_index_00, edge_index_01, edge_index_10, edge_index_11, W_l_00, b_l_00, W_r_00, W_l_01, b_l_01, W_r_01, W_l_10, b_l_10, W_r_10, W_l_11, b_l_11, W_r_11, W_lin_0, b_lin_0, W_lin_1, b_lin_1):
    raise NotImplementedError("write your pallas kernel here")


# speedup vs baseline: 15.1335x; 15.1335x over previous
import functools

import jax
import jax.numpy as jnp
from jax import lax
from jax.experimental import pallas as pl
from jax.experimental.pallas import tpu as pltpu
from jax.experimental.pallas import tpu_sc as plsc

_N = 10000
_D = 128
_E = 320000
_H = 8
_OUT = 128
_R = 4
_NC = 2
_NS = 16
_NW = _NC * _NS
_BATCH = 125
_EPW = _R * _E // _NW
_NB = _EPW // _BATCH
_NP = 10240
_RPC = 2
_ZROWS = 640
_RPS = _RPC * _NP // _NS


def _tables_body(x0_ref, x1_ref, wl_ref, t_ref):
    col = lax.broadcasted_iota(jnp.int32, (_N, 16), 1)
    cnt_col = jnp.where(col == _H, 1.0, 0.0).astype(jnp.float32)
    for r in range(_R):
        x = x0_ref[...] if r < 2 else x1_ref[...]
        p = jnp.dot(x, wl_ref[r], preferred_element_type=jnp.float32)
        t_ref[pl.ds(r * _NP, _N), :] = p + cnt_col


def _edge_body(src_hbm, dst_hbm, tab_hbm, out_hbm,
               src_v, dst_v, rows_v, zero_v, agg_sh, gsem):
    cid = lax.axis_index("c")
    sid = lax.axis_index("s")
    wid = cid * _NS + sid

    def zbody(i, c):
        zero_v[i, :] = jnp.zeros((16,), jnp.float32)
        return c
    lax.fori_loop(0, _ZROWS, zbody, 0)
    base = sid * _RPS
    for k in range(_RPS // _ZROWS):
        pltpu.sync_copy(zero_v, agg_sh.at[pl.ds(base + k * _ZROWS, _ZROWS)])
    plsc.subcore_barrier()

    pltpu.sync_copy(src_hbm.at[wid], src_v)
    pltpu.sync_copy(dst_hbm.at[wid], dst_v)

    def body(j, c):
        pltpu.async_copy(tab_hbm.at[src_v.at[j]], rows_v, gsem).wait()
        pltpu.sync_copy(rows_v, agg_sh.at[dst_v.at[j]], add=True)
        return c
    lax.fori_loop(0, _NB, body, 0)

    plsc.subcore_barrier()
    pltpu.sync_copy(agg_sh.at[pl.ds(base, _RPS)],
                    out_hbm.at[cid].at[pl.ds(base, _RPS)])


@functools.cache
def _edge_kernel():
    return pl.kernel(
        _edge_body,
        out_type=jax.ShapeDtypeStruct((_NC, _RPC * _NP, 16), jnp.float32),
        mesh=plsc.VectorSubcoreMesh(core_axis_name="c", subcore_axis_name="s",
                                    num_cores=_NC, num_subcores=_NS),
        scratch_types=[
            pltpu.VMEM((_NB, _BATCH), jnp.int32),
            pltpu.VMEM((_NB, _BATCH), jnp.int32),
            pltpu.VMEM((_BATCH, 16), jnp.float32),
            pltpu.VMEM((_ZROWS, 16), jnp.float32),
            pltpu.VMEM_SHARED((_RPC * _NP, 16), jnp.float32),
            pltpu.SemaphoreType.DMA,
        ],
        compiler_params=pltpu.CompilerParams(use_tc_tiling_on_sc=False),
    )


def _combine_body(agg_ref, x0_ref, x1_ref, wr_ref, wlin_ref, bl_ref, blin_ref,
                  o0_ref, o1_ref):
    for t in range(2):
        x = x0_ref[...] if t == 0 else x1_ref[...]
        o_ref = o0_ref if t == 0 else o1_ref
        ms = []
        for jp in range(2):
            r = 2 * jp + t
            plane = agg_ref[r // 2][(r % 2) * _NP:(r % 2) * _NP + _N, :]
            cnt = plane[:, _H:_H + 1]
            ms.append(plane[:, :_H] / jnp.maximum(cnt, 1.0))
        cat = jnp.concatenate(ms, axis=1)
        wlin = wlin_ref[t]
        acc = jnp.dot(cat, wlin, preferred_element_type=jnp.float32)
        rm = (jnp.dot(wr_ref[t], wlin[:_H], preferred_element_type=jnp.float32)
              + jnp.dot(wr_ref[2 + t], wlin[_H:],
                        preferred_element_type=jnp.float32))
        acc = acc + jnp.dot(x, rm, preferred_element_type=jnp.float32)
        cvec = (jnp.dot(bl_ref[pl.ds(t, 1), :], wlin[:_H],
                        preferred_element_type=jnp.float32)
                + jnp.dot(bl_ref[pl.ds(2 + t, 1), :], wlin[_H:],
                          preferred_element_type=jnp.float32)
                + blin_ref[pl.ds(t, 1), :])
        o_ref[...] = acc + cvec


def kernel(x_0, x_1, edge_index_00, edge_index_01, edge_index_10,
           edge_index_11, W_l_00, b_l_00, W_r_00, W_l_01, b_l_01, W_r_01,
           W_l_10, b_l_10, W_r_10, W_l_11, b_l_11, W_r_11,
           W_lin_0, b_lin_0, W_lin_1, b_lin_1):
    wl = jnp.stack([W_l_00, W_l_01, W_l_10, W_l_11])
    wl16 = jnp.concatenate([wl, jnp.zeros((_R, _D, 16 - _H), jnp.float32)],
                           axis=2)
    tab = pl.pallas_call(
        _tables_body,
        out_shape=jax.ShapeDtypeStruct((_R * _NP, 16), jnp.float32),
    )(x_0, x_1, wl16)

    goffs = (jnp.arange(_R, dtype=jnp.int32) * _NP)[:, None]
    loffs = ((jnp.arange(_R, dtype=jnp.int32) % _RPC) * _NP)[:, None]
    eis = [edge_index_00, edge_index_01, edge_index_10, edge_index_11]
    src = (jnp.stack([e[0] for e in eis]) + goffs).reshape(_NW, _NB, _BATCH)
    dst = (jnp.stack([e[1] for e in eis]) + loffs).reshape(_NW, _NB, _BATCH)

    agg = _edge_kernel()(src, dst, tab)

    wr = jnp.stack([W_r_00, W_r_01, W_r_10, W_r_11])
    wlin = jnp.stack([W_lin_0, W_lin_1])
    bl = jnp.stack([b_l_00, b_l_01, b_l_10, b_l_11])
    blin = jnp.stack([b_lin_0, b_lin_1])
    out0, out1 = pl.pallas_call(
        _combine_body,
        out_shape=(jax.ShapeDtypeStruct((_N, _OUT), jnp.float32),
                   jax.ShapeDtypeStruct((_N, _OUT), jnp.float32)),
    )(agg, x_0, x_1, wr, wlin, bl, blin)
    return out0, out1


# speedup vs baseline: 30.9933x; 2.0480x over previous
import functools

import jax
import jax.numpy as jnp
from jax import lax
from jax.experimental import pallas as pl
from jax.experimental.pallas import tpu as pltpu
from jax.experimental.pallas import tpu_sc as plsc

_N = 10000
_D = 128
_E = 320000
_H = 8
_OUT = 128
_R = 4
_NC = 2
_NS = 16
_NW = _NC * _NS
_BATCH = 125
_EPW = _R * _E // _NW
_NB = _EPW // _BATCH
_NP = 10240
_RPC = 2
_ZROWS = 128
_RPS = _RPC * _NP // _NS
_SB = 5
_NSUP = _NB // _SB


def _tables_body(x0_ref, x1_ref, wl_ref, t_ref):
    col = lax.broadcasted_iota(jnp.int32, (_N, 16), 1)
    cnt_col = jnp.where(col == _H, 1.0, 0.0).astype(jnp.float32)
    for r in range(_R):
        x = x0_ref[...] if r < 2 else x1_ref[...]
        p = jnp.dot(x, wl_ref[r], preferred_element_type=jnp.float32)
        t_ref[pl.ds(r * _NP, _N), :] = p + cnt_col


def _edge_body(src_hbm, dst_hbm, tab_hbm, out_hbm,
               src_v, dst_v, rows_v, zero_v, agg_sh, gsem, ssem):
    cid = lax.axis_index("c")
    sid = lax.axis_index("s")
    wid = cid * _NS + sid

    def zbody(i, c):
        zero_v[i, :] = jnp.zeros((16,), jnp.float32)
        return c
    lax.fori_loop(0, _ZROWS, zbody, 0)
    base = sid * _RPS
    for k in range(_RPS // _ZROWS):
        pltpu.sync_copy(zero_v, agg_sh.at[pl.ds(base + k * _ZROWS, _ZROWS)])
    plsc.subcore_barrier()

    pltpu.sync_copy(src_hbm.at[wid], src_v)
    pltpu.sync_copy(dst_hbm.at[wid], dst_v)

    def gather_desc(sp, b, p):
        return pltpu.make_async_copy(tab_hbm.at[src_v.at[sp * _SB + b]],
                                     rows_v.at[p, b], gsem.at[p])

    def scatter_desc(sp, b, p):
        return pltpu.make_async_copy(rows_v.at[p, b],
                                     agg_sh.at[dst_v.at[sp * _SB + b]],
                                     ssem.at[p])

    def run_super(sp, p, fire_next):
        for b in range(_SB):
            gather_desc(sp, b, p).wait()
        for b in range(_SB):
            scatter_desc(sp, b, p).start(add=True)
        for b in range(_SB):
            scatter_desc(sp, b, p).wait()
        if fire_next:
            for b in range(_SB):
                gather_desc(sp + 2, b, p).start()

    for p in range(2):
        for b in range(_SB):
            gather_desc(p, b, p).start()

    def body(g, c):
        run_super(2 * g, 0, True)
        run_super(2 * g + 1, 1, True)
        return c
    lax.fori_loop(0, _NSUP // 2 - 1, body, 0)
    run_super(_NSUP - 2, 0, False)
    run_super(_NSUP - 1, 1, False)

    plsc.subcore_barrier()
    pltpu.sync_copy(agg_sh.at[pl.ds(base, _RPS)],
                    out_hbm.at[cid].at[pl.ds(base, _RPS)])


@functools.cache
def _edge_kernel():
    return pl.kernel(
        _edge_body,
        out_type=jax.ShapeDtypeStruct((_NC, _RPC * _NP, 16), jnp.float32),
        mesh=plsc.VectorSubcoreMesh(core_axis_name="c", subcore_axis_name="s",
                                    num_cores=_NC, num_subcores=_NS),
        scratch_types=[
            pltpu.VMEM((_NB, _BATCH), jnp.int32),
            pltpu.VMEM((_NB, _BATCH), jnp.int32),
            pltpu.VMEM((2, _SB, _BATCH, 16), jnp.float32),
            pltpu.VMEM((_ZROWS, 16), jnp.float32),
            pltpu.VMEM_SHARED((_RPC * _NP, 16), jnp.float32),
            pltpu.SemaphoreType.DMA((2,)),
            pltpu.SemaphoreType.DMA((2,)),
        ],
        compiler_params=pltpu.CompilerParams(use_tc_tiling_on_sc=False),
    )


def _combine_body(agg_ref, x0_ref, x1_ref, wr_ref, wlin_ref, bl_ref, blin_ref,
                  o0_ref, o1_ref):
    for t in range(2):
        x = x0_ref[...] if t == 0 else x1_ref[...]
        o_ref = o0_ref if t == 0 else o1_ref
        ms = []
        for jp in range(2):
            r = 2 * jp + t
            plane = agg_ref[r // 2][(r % 2) * _NP:(r % 2) * _NP + _N, :]
            cnt = plane[:, _H:_H + 1]
            ms.append(plane[:, :_H] / jnp.maximum(cnt, 1.0))
        cat = jnp.concatenate(ms, axis=1)
        wlin = wlin_ref[t]
        acc = jnp.dot(cat, wlin, preferred_element_type=jnp.float32)
        rm = (jnp.dot(wr_ref[t], wlin[:_H], preferred_element_type=jnp.float32)
              + jnp.dot(wr_ref[2 + t], wlin[_H:],
                        preferred_element_type=jnp.float32))
        acc = acc + jnp.dot(x, rm, preferred_element_type=jnp.float32)
        cvec = (jnp.dot(bl_ref[pl.ds(t, 1), :], wlin[:_H],
                        preferred_element_type=jnp.float32)
                + jnp.dot(bl_ref[pl.ds(2 + t, 1), :], wlin[_H:],
                          preferred_element_type=jnp.float32)
                + blin_ref[pl.ds(t, 1), :])
        o_ref[...] = acc + cvec


def kernel(x_0, x_1, edge_index_00, edge_index_01, edge_index_10,
           edge_index_11, W_l_00, b_l_00, W_r_00, W_l_01, b_l_01, W_r_01,
           W_l_10, b_l_10, W_r_10, W_l_11, b_l_11, W_r_11,
           W_lin_0, b_lin_0, W_lin_1, b_lin_1):
    wl = jnp.stack([W_l_00, W_l_01, W_l_10, W_l_11])
    wl16 = jnp.concatenate([wl, jnp.zeros((_R, _D, 16 - _H), jnp.float32)],
                           axis=2)
    tab = pl.pallas_call(
        _tables_body,
        out_shape=jax.ShapeDtypeStruct((_R * _NP, 16), jnp.float32),
    )(x_0, x_1, wl16)

    goffs = (jnp.arange(_R, dtype=jnp.int32) * _NP)[:, None]
    loffs = ((jnp.arange(_R, dtype=jnp.int32) % _RPC) * _NP)[:, None]
    eis = [edge_index_00, edge_index_01, edge_index_10, edge_index_11]
    src = (jnp.stack([e[0] for e in eis]) + goffs).reshape(_NW, _NB, _BATCH)
    dst = (jnp.stack([e[1] for e in eis]) + loffs).reshape(_NW, _NB, _BATCH)

    agg = _edge_kernel()(src, dst, tab)

    wr = jnp.stack([W_r_00, W_r_01, W_r_10, W_r_11])
    wlin = jnp.stack([W_lin_0, W_lin_1])
    bl = jnp.stack([b_l_00, b_l_01, b_l_10, b_l_11])
    blin = jnp.stack([b_lin_0, b_lin_1])
    out0, out1 = pl.pallas_call(
        _combine_body,
        out_shape=(jax.ShapeDtypeStruct((_N, _OUT), jnp.float32),
                   jax.ShapeDtypeStruct((_N, _OUT), jnp.float32)),
    )(agg, x_0, x_1, wr, wlin, bl, blin)
    return out0, out1


# speedup vs baseline: 34.6723x; 1.1187x over previous
import functools

import jax
import jax.numpy as jnp
from jax import lax
from jax.experimental import pallas as pl
from jax.experimental.pallas import tpu as pltpu
from jax.experimental.pallas import tpu_sc as plsc

_N = 10000
_D = 128
_E = 320000
_H = 8
_OUT = 128
_R = 4
_NC = 2
_NS = 16
_NW = _NC * _NS
_BATCH = 125
_EROWS = _E // _BATCH
_NB = _EROWS // 8
_NP = 10240
_RPC = 2
_ZROWS = 128
_PPS = _NP // _NS
_SB = 5
_NSUP = _NB // _SB
_BN = 1000


def _tables_body(x0_ref, x1_ref, wl_ref, t_ref):
    col = lax.broadcasted_iota(jnp.int32, (_BN, 16), 1)
    cnt_col = jnp.where(col == _H, 1.0, 0.0).astype(jnp.float32)
    for r in range(_R):
        x = x0_ref[...] if r < 2 else x1_ref[...]
        p = jnp.dot(x, wl_ref[r], preferred_element_type=jnp.float32)
        t_ref[r] = p + cnt_col


def _edge_body(e00, e01, e10, e11, tab_hbm, out_hbm,
               src_v, dst_v, rows_v, zero_v, agg_sh, gsem, ssem):
    cid = lax.axis_index("c")
    sid = lax.axis_index("s")
    wid = cid * _NS + sid
    rel = wid // 8
    lrel = rel % 2
    row0 = (wid % 8) * _NB

    def zbody(i, c):
        zero_v[i, :] = jnp.zeros((16,), jnp.float32)
        return c
    lax.fori_loop(0, _ZROWS, zbody, 0)
    pbase = sid * _PPS
    for q in range(_RPC):
        for k in range(_PPS // _ZROWS):
            pltpu.sync_copy(zero_v,
                            agg_sh.at[q].at[pl.ds(pbase + k * _ZROWS, _ZROWS)])
    plsc.subcore_barrier()

    for r, e in enumerate((e00, e01, e10, e11)):
        @pl.when(rel == r)
        def _():
            pltpu.sync_copy(e.at[0].at[pl.ds(row0, _NB)], src_v)
            pltpu.sync_copy(e.at[1].at[pl.ds(row0, _NB)], dst_v)

    def gather_desc(sp, b, p):
        return pltpu.make_async_copy(
            tab_hbm.at[rel].at[src_v.at[sp * _SB + b]],
            rows_v.at[p, b], gsem.at[p])

    def scatter_desc(sp, b, p):
        return pltpu.make_async_copy(
            rows_v.at[p, b],
            agg_sh.at[lrel].at[dst_v.at[sp * _SB + b]], ssem.at[p])

    def run_super(sp, p, fire_next):
        for b in range(_SB):
            gather_desc(sp, b, p).wait()
        for b in range(_SB):
            scatter_desc(sp, b, p).start(add=True)
        for b in range(_SB):
            scatter_desc(sp, b, p).wait()
        if fire_next:
            for b in range(_SB):
                gather_desc(sp + 2, b, p).start()

    for p in range(2):
        for b in range(_SB):
            gather_desc(p, b, p).start()

    def body(g, c):
        run_super(2 * g, 0, True)
        run_super(2 * g + 1, 1, True)
        return c
    lax.fori_loop(0, _NSUP // 2 - 1, body, 0)
    run_super(_NSUP - 2, 0, False)
    run_super(_NSUP - 1, 1, False)

    plsc.subcore_barrier()
    for q in range(_RPC):
        pltpu.sync_copy(agg_sh.at[q].at[pl.ds(pbase, _PPS)],
                        out_hbm.at[cid].at[q].at[pl.ds(pbase, _PPS)])


@functools.cache
def _edge_kernel():
    return pl.kernel(
        _edge_body,
        out_type=jax.ShapeDtypeStruct((_NC, _RPC, _NP, 16), jnp.float32),
        mesh=plsc.VectorSubcoreMesh(core_axis_name="c", subcore_axis_name="s",
                                    num_cores=_NC, num_subcores=_NS),
        scratch_types=[
            pltpu.VMEM((_NB, _BATCH), jnp.int32),
            pltpu.VMEM((_NB, _BATCH), jnp.int32),
            pltpu.VMEM((2, _SB, _BATCH, 16), jnp.float32),
            pltpu.VMEM((_ZROWS, 16), jnp.float32),
            pltpu.VMEM_SHARED((_RPC, _NP, 16), jnp.float32),
            pltpu.SemaphoreType.DMA((2,)),
            pltpu.SemaphoreType.DMA((2,)),
        ],
        compiler_params=pltpu.CompilerParams(use_tc_tiling_on_sc=False),
    )


def _combine_body(agg_ref, x0_ref, x1_ref, wr_ref, wlin_ref, bl_ref, blin_ref,
                  o0_ref, o1_ref):
    for t in range(2):
        x = x0_ref[...] if t == 0 else x1_ref[...]
        o_ref = o0_ref if t == 0 else o1_ref
        ms = []
        for jp in range(2):
            plane = agg_ref[jp, t]
            cnt = plane[:, _H:_H + 1]
            ms.append(plane[:, :_H] / jnp.maximum(cnt, 1.0))
        cat = jnp.concatenate(ms, axis=1)
        wlin = wlin_ref[t]
        acc = jnp.dot(cat, wlin, preferred_element_type=jnp.float32)
        rm = (jnp.dot(wr_ref[t], wlin[:_H], preferred_element_type=jnp.float32)
              + jnp.dot(wr_ref[2 + t], wlin[_H:],
                        preferred_element_type=jnp.float32))
        acc = acc + jnp.dot(x, rm, preferred_element_type=jnp.float32)
        cvec = (jnp.dot(bl_ref[pl.ds(t, 1), :], wlin[:_H],
                        preferred_element_type=jnp.float32)
                + jnp.dot(bl_ref[pl.ds(2 + t, 1), :], wlin[_H:],
                          preferred_element_type=jnp.float32)
                + blin_ref[pl.ds(t, 1), :])
        o_ref[...] = acc + cvec


def kernel(x_0, x_1, edge_index_00, edge_index_01, edge_index_10,
           edge_index_11, W_l_00, b_l_00, W_r_00, W_l_01, b_l_01, W_r_01,
           W_l_10, b_l_10, W_r_10, W_l_11, b_l_11, W_r_11,
           W_lin_0, b_lin_0, W_lin_1, b_lin_1):
    nblk = _N // _BN
    wl = jnp.stack([W_l_00, W_l_01, W_l_10, W_l_11])
    wl16 = jnp.concatenate([wl, jnp.zeros((_R, _D, 16 - _H), jnp.float32)],
                           axis=2)
    tab = pl.pallas_call(
        _tables_body,
        grid=(nblk,),
        in_specs=[
            pl.BlockSpec((_BN, _D), lambda i: (i, 0)),
            pl.BlockSpec((_BN, _D), lambda i: (i, 0)),
            pl.BlockSpec((_R, _D, 16), lambda i: (0, 0, 0)),
        ],
        out_specs=pl.BlockSpec((_R, _BN, 16), lambda i: (0, i, 0)),
        out_shape=jax.ShapeDtypeStruct((_R, _NP, 16), jnp.float32),
    )(x_0, x_1, wl16)

    eis = [e.reshape(2, _EROWS, _BATCH) for e in
           (edge_index_00, edge_index_01, edge_index_10, edge_index_11)]
    agg = _edge_kernel()(*eis, tab)

    wr = jnp.stack([W_r_00, W_r_01, W_r_10, W_r_11])
    wlin = jnp.stack([W_lin_0, W_lin_1])
    bl = jnp.stack([b_l_00, b_l_01, b_l_10, b_l_11])
    blin = jnp.stack([b_lin_0, b_lin_1])
    out0, out1 = pl.pallas_call(
        _combine_body,
        grid=(nblk,),
        in_specs=[
            pl.BlockSpec((_NC, _RPC, _BN, 16), lambda i: (0, 0, i, 0)),
            pl.BlockSpec((_BN, _D), lambda i: (i, 0)),
            pl.BlockSpec((_BN, _D), lambda i: (i, 0)),
            pl.BlockSpec((_R, _D, _H), lambda i: (0, 0, 0)),
            pl.BlockSpec((2, 16, _OUT), lambda i: (0, 0, 0)),
            pl.BlockSpec((_R, _H), lambda i: (0, 0)),
            pl.BlockSpec((2, _OUT), lambda i: (0, 0)),
        ],
        out_specs=(pl.BlockSpec((_BN, _OUT), lambda i: (i, 0)),
                   pl.BlockSpec((_BN, _OUT), lambda i: (i, 0))),
        out_shape=(jax.ShapeDtypeStruct((_N, _OUT), jnp.float32),
                   jax.ShapeDtypeStruct((_N, _OUT), jnp.float32)),
    )(agg, x_0, x_1, wr, wlin, bl, blin)
    return out0, out1


# speedup vs baseline: 45.4617x; 1.3112x over previous
import functools

import jax
import jax.numpy as jnp
from jax import lax
from jax.experimental import pallas as pl
from jax.experimental.pallas import tpu as pltpu
from jax.experimental.pallas import tpu_sc as plsc

_N = 10000
_D = 128
_E = 320000
_H = 8
_OUT = 128
_R = 4
_NC = 2
_NS = 16
_NW = _NC * _NS
_BATCH = 128
_EROWS = _E // _BATCH
_WROWS = _EROWS // 8
_TROWS = _EROWS - 8 * _WROWS
_NP = 10240
_RPC = 2
_ZROWS = 128
_PPS = _NP // _NS
_SB = 6
_NSUP = _WROWS // _SB
_BN = 1000


def _tables_body(x0_ref, x1_ref, wl_ref, t_ref):
    col = lax.broadcasted_iota(jnp.int32, (_BN, 16), 1)
    cnt_col = jnp.where(col == _H, 1.0, 0.0).astype(jnp.float32)
    for r in range(_R):
        x = x0_ref[...] if r < 2 else x1_ref[...]
        p = jnp.dot(x, wl_ref[r], preferred_element_type=jnp.float32)
        t_ref[r] = p + cnt_col


def _edge_body(e00, e01, e10, e11, tab_hbm, out_hbm,
               ebuf, tbuf, rows_v, zero_v, agg_sh, gsem, ssem):
    cid = lax.axis_index("c")
    sid = lax.axis_index("s")
    wid = cid * _NS + sid
    rel = wid // 8
    lrel = rel % 2
    row0 = (wid % 8) * _WROWS

    def zbody(i, c):
        zero_v[i, :] = jnp.zeros((16,), jnp.float32)
        return c
    lax.fori_loop(0, _ZROWS, zbody, 0)
    pbase = sid * _PPS
    for q in range(_RPC):
        for k in range(_PPS // _ZROWS):
            pltpu.sync_copy(zero_v,
                            agg_sh.at[q].at[pl.ds(pbase + k * _ZROWS, _ZROWS)])
    plsc.subcore_barrier()

    for r, e in enumerate((e00, e01, e10, e11)):
        @pl.when(rel == r)
        def _():
            pltpu.sync_copy(e.at[pl.ds(row0, _WROWS)], ebuf)
            @pl.when(wid % 8 == 0)
            def _():
                pltpu.sync_copy(e.at[pl.ds(8 * _WROWS, _TROWS)], tbuf)

    def gather_desc(sp, b, p):
        return pltpu.make_async_copy(
            tab_hbm.at[rel].at[ebuf.at[sp * _SB + b, 0]],
            rows_v.at[p, b], gsem.at[p])

    def scatter_desc(sp, b, p):
        return pltpu.make_async_copy(
            rows_v.at[p, b],
            agg_sh.at[lrel].at[ebuf.at[sp * _SB + b, 1]], ssem.at[p])

    def run_super(sp, p, fire_next):
        for b in range(_SB):
            gather_desc(sp, b, p).wait()
        for b in range(_SB):
            scatter_desc(sp, b, p).start(add=True)
        for b in range(_SB):
            scatter_desc(sp, b, p).wait()
        if fire_next:
            for b in range(_SB):
                gather_desc(sp + 2, b, p).start()

    for p in range(2):
        for b in range(_SB):
            gather_desc(p, b, p).start()

    def body(g, c):
        run_super(2 * g, 0, True)
        run_super(2 * g + 1, 1, True)
        return c
    lax.fori_loop(0, _NSUP // 2 - 1, body, 0)
    run_super(_NSUP - 2, 0, False)
    run_super(_NSUP - 1, 1, False)

    @pl.when(wid % 8 == 0)
    def _():
        for b in range(_TROWS):
            pltpu.async_copy(tab_hbm.at[rel].at[tbuf.at[b, 0]],
                             rows_v.at[0, 0], gsem.at[0]).wait()
            pltpu.sync_copy(rows_v.at[0, 0],
                            agg_sh.at[lrel].at[tbuf.at[b, 1]], add=True)

    plsc.subcore_barrier()
    for q in range(_RPC):
        pltpu.sync_copy(
            agg_sh.at[q].at[pl.ds(pbase, _PPS)],
            out_hbm.at[cid].at[q].at[pl.ds(pbase, _PPS), pl.ds(0, 16)])


@functools.cache
def _edge_kernel():
    return pl.kernel(
        _edge_body,
        out_type=jax.ShapeDtypeStruct((_NC, _RPC, _NP, 128), jnp.float32),
        mesh=plsc.VectorSubcoreMesh(core_axis_name="c", subcore_axis_name="s",
                                    num_cores=_NC, num_subcores=_NS),
        scratch_types=[
            pltpu.VMEM((_WROWS, 2, _BATCH), jnp.int32),
            pltpu.VMEM((_TROWS, 2, _BATCH), jnp.int32),
            pltpu.VMEM((2, _SB, _BATCH, 16), jnp.float32),
            pltpu.VMEM((_ZROWS, 16), jnp.float32),
            pltpu.VMEM_SHARED((_RPC, _NP, 16), jnp.float32),
            pltpu.SemaphoreType.DMA((2,)),
            pltpu.SemaphoreType.DMA((2,)),
        ],
        compiler_params=pltpu.CompilerParams(use_tc_tiling_on_sc=False),
    )


def _combine_body(agg0_ref, agg1_ref, x0_ref, x1_ref, wr_ref, wlin_ref,
                  bl_ref, blin_ref, o0_ref, o1_ref):
    for t in range(2):
        agg_ref = agg0_ref if t == 0 else agg1_ref
        x = x0_ref[...] if t == 0 else x1_ref[...]
        o_ref = o0_ref if t == 0 else o1_ref
        ms = []
        for jp in range(2):
            plane = agg_ref[jp, 0][:, :16]
            cnt = plane[:, _H:_H + 1]
            ms.append(plane[:, :_H] / jnp.maximum(cnt, 1.0))
        cat = jnp.concatenate(ms, axis=1)
        wlin = wlin_ref[t]
        acc = jnp.dot(cat, wlin, preferred_element_type=jnp.float32)
        rm = (jnp.dot(wr_ref[t], wlin[:_H], preferred_element_type=jnp.float32)
              + jnp.dot(wr_ref[2 + t], wlin[_H:],
                        preferred_element_type=jnp.float32))
        acc = acc + jnp.dot(x, rm, preferred_element_type=jnp.float32)
        cvec = (jnp.dot(bl_ref[pl.ds(t, 1), :], wlin[:_H],
                        preferred_element_type=jnp.float32)
                + jnp.dot(bl_ref[pl.ds(2 + t, 1), :], wlin[_H:],
                          preferred_element_type=jnp.float32)
                + blin_ref[pl.ds(t, 1), :])
        o_ref[...] = acc + cvec


def kernel(x_0, x_1, edge_index_00, edge_index_01, edge_index_10,
           edge_index_11, W_l_00, b_l_00, W_r_00, W_l_01, b_l_01, W_r_01,
           W_l_10, b_l_10, W_r_10, W_l_11, b_l_11, W_r_11,
           W_lin_0, b_lin_0, W_lin_1, b_lin_1):
    nblk = _N // _BN
    wl = jnp.stack([W_l_00, W_l_01, W_l_10, W_l_11])
    wl16 = jnp.concatenate([wl, jnp.zeros((_R, _D, 16 - _H), jnp.float32)],
                           axis=2)
    tab = pl.pallas_call(
        _tables_body,
        grid=(nblk,),
        in_specs=[
            pl.BlockSpec((_BN, _D), lambda i: (i, 0)),
            pl.BlockSpec((_BN, _D), lambda i: (i, 0)),
            pl.BlockSpec((_R, _D, 16), lambda i: (0, 0, 0)),
        ],
        out_specs=pl.BlockSpec((_R, _BN, 16), lambda i: (0, i, 0)),
        out_shape=jax.ShapeDtypeStruct((_R, _NP, 16), jnp.float32),
    )(x_0, x_1, wl16)

    eis = [e.reshape(2, _EROWS, _BATCH).transpose(1, 0, 2) for e in
           (edge_index_00, edge_index_01, edge_index_10, edge_index_11)]
    agg = _edge_kernel()(*eis, tab)

    wr = jnp.stack([W_r_00, W_r_01, W_r_10, W_r_11])
    wlin = jnp.stack([W_lin_0, W_lin_1])
    bl = jnp.stack([b_l_00, b_l_01, b_l_10, b_l_11])
    blin = jnp.stack([b_lin_0, b_lin_1])
    out0, out1 = pl.pallas_call(
        _combine_body,
        grid=(nblk,),
        in_specs=[
            pl.BlockSpec((_NC, 1, _BN, 128), lambda i: (0, 0, i, 0)),
            pl.BlockSpec((_NC, 1, _BN, 128), lambda i: (0, 1, i, 0)),
            pl.BlockSpec((_BN, _D), lambda i: (i, 0)),
            pl.BlockSpec((_BN, _D), lambda i: (i, 0)),
            pl.BlockSpec((_R, _D, _H), lambda i: (0, 0, 0)),
            pl.BlockSpec((2, 16, _OUT), lambda i: (0, 0, 0)),
            pl.BlockSpec((_R, _H), lambda i: (0, 0)),
            pl.BlockSpec((2, _OUT), lambda i: (0, 0)),
        ],
        out_specs=(pl.BlockSpec((_BN, _OUT), lambda i: (i, 0)),
                   pl.BlockSpec((_BN, _OUT), lambda i: (i, 0))),
        out_shape=(jax.ShapeDtypeStruct((_N, _OUT), jnp.float32),
                   jax.ShapeDtypeStruct((_N, _OUT), jnp.float32)),
    )(agg, agg, x_0, x_1, wr, wlin, bl, blin)
    return out0, out1


# speedup vs baseline: 47.9411x; 1.0545x over previous
import functools

import jax
import jax.numpy as jnp
from jax import lax
from jax.experimental import pallas as pl
from jax.experimental.pallas import tpu as pltpu
from jax.experimental.pallas import tpu_sc as plsc

_N = 10000
_D = 128
_E = 320000
_H = 8
_OUT = 128
_R = 4
_NC = 2
_NS = 16
_NW = _NC * _NS
_BATCH = 128
_EROWS = _E // _BATCH
_WROWS = _EROWS // 8
_TROWS = _EROWS - 8 * _WROWS
_NP = 10240
_RPC = 2
_ZROWS = 128
_PPS = _NP // _NS
_SB = 6
_NSUP = _WROWS // _SB
_XROWS = _N // 8
_BNP = 160
_NBLK = _NP // (8 * _BNP)


def _tables_body(x0_ref, x1_ref, w3_ref, t_ref):
    r = pl.program_id(0)
    col = lax.broadcasted_iota(jnp.int32, (_XROWS, 128), 1)
    cnt_col = jnp.where(col % 16 == _H, 1.0, 0.0).astype(jnp.float32)
    zpad = jnp.zeros((_NP // 8 - _XROWS, 128), jnp.float32)

    def emit(x_ref):
        p = jnp.dot(x_ref[...], w3_ref[0],
                    preferred_element_type=jnp.float32) + cnt_col
        t_ref[0] = jnp.concatenate([p, zpad], axis=0)

    @pl.when(r < 2)
    def _():
        emit(x0_ref)

    @pl.when(r >= 2)
    def _():
        emit(x1_ref)


def _edge_body(e00, e01, e10, e11, tab_hbm, out_hbm,
               ebuf, tbuf, rows_v, zero_v, agg_sh, gsem, ssem):
    cid = lax.axis_index("c")
    sid = lax.axis_index("s")
    wid = cid * _NS + sid
    rel = wid // 8
    lrel = rel % 2
    row0 = (wid % 8) * _WROWS

    def zbody(i, c):
        zero_v[i, :] = jnp.zeros((16,), jnp.float32)
        return c
    lax.fori_loop(0, _ZROWS, zbody, 0)
    pbase = sid * _PPS
    for q in range(_RPC):
        for k in range(_PPS // _ZROWS):
            pltpu.sync_copy(zero_v,
                            agg_sh.at[q].at[pl.ds(pbase + k * _ZROWS, _ZROWS)])
    plsc.subcore_barrier()

    for r, e in enumerate((e00, e01, e10, e11)):
        @pl.when(rel == r)
        def _():
            pltpu.sync_copy(e.at[pl.ds(row0, _WROWS)], ebuf)
            @pl.when(wid % 8 == 0)
            def _():
                pltpu.sync_copy(e.at[pl.ds(8 * _WROWS, _TROWS)], tbuf)

    def gather_desc(sp, b, p):
        return pltpu.make_async_copy(
            tab_hbm.at[rel].at[ebuf.at[sp * _SB + b, 0]],
            rows_v.at[p, b], gsem.at[p])

    def scatter_desc(sp, b, p):
        return pltpu.make_async_copy(
            rows_v.at[p, b],
            agg_sh.at[lrel].at[ebuf.at[sp * _SB + b, 1]], ssem.at[p])

    def run_super(sp, p, fire_next):
        for b in range(_SB):
            gather_desc(sp, b, p).wait()
        for b in range(_SB):
            scatter_desc(sp, b, p).start(add=True)
        for b in range(_SB):
            scatter_desc(sp, b, p).wait()
        if fire_next:
            for b in range(_SB):
                gather_desc(sp + 2, b, p).start()

    for p in range(2):
        for b in range(_SB):
            gather_desc(p, b, p).start()

    def body(g, c):
        run_super(2 * g, 0, True)
        run_super(2 * g + 1, 1, True)
        return c
    lax.fori_loop(0, _NSUP // 2 - 1, body, 0)
    run_super(_NSUP - 2, 0, False)
    run_super(_NSUP - 1, 1, False)

    @pl.when(wid % 8 == 0)
    def _():
        for b in range(_TROWS):
            pltpu.async_copy(tab_hbm.at[rel].at[tbuf.at[b, 0]],
                             rows_v.at[0, 0], gsem.at[0]).wait()
            pltpu.sync_copy(rows_v.at[0, 0],
                            agg_sh.at[lrel].at[tbuf.at[b, 1]], add=True)

    plsc.subcore_barrier()
    for q in range(_RPC):
        pltpu.sync_copy(agg_sh.at[q].at[pl.ds(pbase, _PPS)],
                        out_hbm.at[cid].at[q].at[pl.ds(pbase, _PPS)])


@functools.cache
def _edge_kernel():
    return pl.kernel(
        _edge_body,
        out_type=jax.ShapeDtypeStruct((_NC, _RPC, _NP, 16), jnp.float32),
        mesh=plsc.VectorSubcoreMesh(core_axis_name="c", subcore_axis_name="s",
                                    num_cores=_NC, num_subcores=_NS),
        scratch_types=[
            pltpu.VMEM((_WROWS, 2, _BATCH), jnp.int32),
            pltpu.VMEM((_TROWS, 2, _BATCH), jnp.int32),
            pltpu.VMEM((2, _SB, _BATCH, 16), jnp.float32),
            pltpu.VMEM((_ZROWS, 16), jnp.float32),
            pltpu.VMEM_SHARED((_RPC, _NP, 16), jnp.float32),
            pltpu.SemaphoreType.DMA((2,)),
            pltpu.SemaphoreType.DMA((2,)),
        ],
        compiler_params=pltpu.CompilerParams(use_tc_tiling_on_sc=False),
    )


def _combine_body(agg0_ref, agg1_ref, x0_ref, x1_ref, wr_ref, wlin_ref,
                  w2_ref, bl_ref, blin_ref, o0_ref, o1_ref):
    lcol = lax.broadcasted_iota(jnp.int32, (128, 128), 0)
    jcol = lax.broadcasted_iota(jnp.int32, (128, 128), 1)
    sel = jnp.where(lcol == (jcol // 16) * 16 + _H, 1.0, 0.0)
    sel = sel.astype(jnp.float32)
    for t in range(2):
        agg_ref = agg0_ref if t == 0 else agg1_ref
        xv = x0_ref[...] if t == 0 else x1_ref[...]
        o_ref = o0_ref if t == 0 else o1_ref
        wlin = wlin_ref[t]
        acc = None
        for jp in range(2):
            p = agg_ref[jp, 0]
            cntb = jnp.dot(p, sel, preferred_element_type=jnp.float32)
            m = p / jnp.maximum(cntb, 1.0)
            term = jnp.dot(m, w2_ref[t, jp],
                           preferred_element_type=jnp.float32)
            acc = term if acc is None else acc + term
        rm = (jnp.dot(wr_ref[t], wlin[:_H], preferred_element_type=jnp.float32)
              + jnp.dot(wr_ref[2 + t], wlin[_H:],
                        preferred_element_type=jnp.float32))
        xr = jnp.concatenate(
            [jnp.dot(xv[:, 128 * u:128 * (u + 1)], rm,
                     preferred_element_type=jnp.float32) for u in range(8)],
            axis=1)
        cvec = (jnp.dot(bl_ref[pl.ds(t, 1), :], wlin[:_H],
                        preferred_element_type=jnp.float32)
                + jnp.dot(bl_ref[pl.ds(2 + t, 1), :], wlin[_H:],
                          preferred_element_type=jnp.float32)
                + blin_ref[pl.ds(t, 1), :])
        cpack = jnp.concatenate([cvec] * 8, axis=1)
        o_ref[...] = acc + xr + cpack


def kernel(x_0, x_1, edge_index_00, edge_index_01, edge_index_10,
           edge_index_11, W_l_00, b_l_00, W_r_00, W_l_01, b_l_01, W_r_01,
           W_l_10, b_l_10, W_r_10, W_l_11, b_l_11, W_r_11,
           W_lin_0, b_lin_0, W_lin_1, b_lin_1):
    eye8 = jnp.eye(8, dtype=jnp.float32)
    wl = jnp.stack([W_l_00, W_l_01, W_l_10, W_l_11])
    wlx = jnp.concatenate([wl, jnp.zeros((_R, _D, 16 - _H), jnp.float32)], 2)
    w3 = jnp.einsum('ab,rkc->rakbc', eye8, wlx).reshape(_R, 1024, 128)
    wlin = jnp.stack([W_lin_0, W_lin_1])
    w2 = jnp.stack([
        jnp.stack([
            jnp.einsum('ab,co->acbo', eye8,
                       jnp.pad(wlin[t, 8 * jp:8 * jp + 8],
                               ((0, 8), (0, 0)))).reshape(128, 1024)
            for jp in range(2)])
        for t in range(2)])

    x0v = x_0.reshape(_N // 8, 8 * _D)
    x1v = x_1.reshape(_N // 8, 8 * _D)
    tabp = pl.pallas_call(
        _tables_body,
        grid=(_R,),
        in_specs=[
            pl.BlockSpec((_XROWS, 8 * _D), lambda r: (0, 0)),
            pl.BlockSpec((_XROWS, 8 * _D), lambda r: (0, 0)),
            pl.BlockSpec((1, 1024, 128), lambda r: (r, 0, 0)),
        ],
        out_specs=pl.BlockSpec((1, _NP // 8, 128), lambda r: (r, 0, 0)),
        out_shape=jax.ShapeDtypeStruct((_R, _NP // 8, 128), jnp.float32),
    )(x0v, x1v, w3)
    tab = tabp.reshape(_R, _NP, 16)

    eis = [e.reshape(2, _EROWS, _BATCH).transpose(1, 0, 2) for e in
           (edge_index_00, edge_index_01, edge_index_10, edge_index_11)]
    agg = _edge_kernel()(*eis, tab)
    aggp = agg.reshape(_NC, _RPC, _NP // 8, 128)

    wr = jnp.stack([W_r_00, W_r_01, W_r_10, W_r_11])
    bl = jnp.stack([b_l_00, b_l_01, b_l_10, b_l_11])
    blin = jnp.stack([b_lin_0, b_lin_1])
    out0p, out1p = pl.pallas_call(
        _combine_body,
        grid=(_NBLK,),
        in_specs=[
            pl.BlockSpec((_NC, 1, _BNP, 128), lambda i: (0, 0, i, 0)),
            pl.BlockSpec((_NC, 1, _BNP, 128), lambda i: (0, 1, i, 0)),
            pl.BlockSpec((_BNP, 8 * _D), lambda i: (i, 0)),
            pl.BlockSpec((_BNP, 8 * _D), lambda i: (i, 0)),
            pl.BlockSpec((_R, _D, _H), lambda i: (0, 0, 0)),
            pl.BlockSpec((2, 16, _OUT), lambda i: (0, 0, 0)),
            pl.BlockSpec((2, 2, 128, 1024), lambda i: (0, 0, 0, 0)),
            pl.BlockSpec((_R, _H), lambda i: (0, 0)),
            pl.BlockSpec((2, _OUT), lambda i: (0, 0)),
        ],
        out_specs=(pl.BlockSpec((_BNP, 8 * _OUT), lambda i: (i, 0)),
                   pl.BlockSpec((_BNP, 8 * _OUT), lambda i: (i, 0))),
        out_shape=(jax.ShapeDtypeStruct((_N // 8, 8 * _OUT), jnp.float32),
                   jax.ShapeDtypeStruct((_N // 8, 8 * _OUT), jnp.float32)),
    )(aggp, aggp, x0v, x1v, wr, wlin, w2, bl, blin)
    return out0p.reshape(_N, _OUT), out1p.reshape(_N, _OUT)
